# baseline (speedup 1.0000x reference)
import functools

import jax
import jax.numpy as jnp
from jax import lax
from jax.experimental import pallas as pl
from jax.experimental.pallas import tpu as pltpu
from jax.experimental.pallas import tpu_sc as plsc

_NC = 2
_NS = 16
_B = 128
_CG = 40
_CW = 8
_ALPHA = 0.5


def _sc_aggregate(x, g_in, s_in, g_out, s_out, n_nodes, n_groups, acc_rows):
    d = x.shape[1]
    rows_per_tile = acc_rows // _NS
    wb = -(-(n_nodes // _NS) // _B) * _B
    n_full = n_nodes // wb
    wb_tail = n_nodes - n_full * wb

    mesh = plsc.VectorSubcoreMesh(
        core_axis_name="c", subcore_axis_name="s",
        num_cores=_NC, num_subcores=_NS)

    @functools.partial(
        pl.kernel,
        out_type=[
            jax.ShapeDtypeStruct((n_nodes, d), jnp.float32),
            jax.ShapeDtypeStruct((n_nodes,), jnp.float32),
            jax.ShapeDtypeStruct((n_nodes, d), jnp.float32),
            jax.ShapeDtypeStruct((n_nodes,), jnp.float32),
        ],
        mesh=mesh,
        scratch_types=[
            pltpu.VMEM_SHARED((acc_rows, d), jnp.float32),
            pltpu.VMEM_SHARED((acc_rows,), jnp.float32),
            pltpu.VMEM((_CG, _B), jnp.int32),
            pltpu.VMEM((_CG, _B), jnp.int32),
            pltpu.VMEM((_B, d), jnp.float32),
            pltpu.VMEM((_B, d), jnp.float32),
            pltpu.VMEM((_CW, _B), jnp.float32),
            pltpu.VMEM((wb,), jnp.float32),
            pltpu.SemaphoreType.DMA,
            pltpu.SemaphoreType.DMA,
            pltpu.SemaphoreType.DMA,
            pltpu.SemaphoreType.DMA,
            pltpu.SemaphoreType.DMA,
        ],
    )
    def agg(x_hbm, gin_hbm, sin_hbm, gout_hbm, sout_hbm,
            sum_in_hbm, cnt_in_hbm, sum_out_hbm, cnt_out_hbm,
            acc_sh, cnt_sh, gi_v, si_v, rows_v, rows2_v, ones_g, cnt_v,
            sem_g0, sem_g1, sem_s0, sem_s1, sem_c):
        c = lax.axis_index("c")
        s = lax.axis_index("s")

        zero16 = jnp.zeros((16,), jnp.float32)
        one16 = jnp.ones((16,), jnp.float32)

        def _zero_row(r, carry):
            for k in range(d // 16):
                rows_v[r, pl.ds(k * 16, 16)] = zero16
            return carry
        lax.fori_loop(0, _B, _zero_row, 0)
        for r in range(_CW):
            for k in range(_B // 16):
                ones_g[r, pl.ds(k * 16, 16)] = one16

        def _zero_acc(k, carry):
            base = s * rows_per_tile + k * _B
            pltpu.sync_copy(rows_v, acc_sh.at[pl.ds(base, _B)])
            pltpu.sync_copy(rows_v.at[0], cnt_sh.at[pl.ds(base, _B)])
            return carry
        lax.fori_loop(0, rows_per_tile // _B, _zero_acc, 0)

        plsc.subcore_barrier()

        def _gather(j, buf, sem):
            pltpu.async_copy(x_hbm.at[gi_v.at[j]], buf, sem)

        def _drain_rows(sem):
            pltpu.make_async_copy(x_hbm.at[gi_v.at[0]], rows_v, sem).wait()

        def _pair(p, carry):
            j0 = 2 * p
            _drain_rows(sem_g0)
            pltpu.async_copy(rows_v, acc_sh.at[si_v.at[j0]], sem_s0, add=True)
            pltpu.async_copy(ones_g.at[0], cnt_sh.at[si_v.at[j0]], sem_c,
                             add=True)
            _drain_rows(sem_g1)
            pltpu.async_copy(rows2_v, acc_sh.at[si_v.at[j0 + 1]], sem_s1,
                             add=True)
            pltpu.async_copy(ones_g.at[0], cnt_sh.at[si_v.at[j0 + 1]], sem_c,
                             add=True)

            @pl.when(p < _CG // 2 - 1)
            def _():
                _drain_rows(sem_s0)
                _gather(j0 + 2, rows_v, sem_g0)
                _drain_rows(sem_s1)
                _gather(j0 + 3, rows2_v, sem_g1)
            return carry

        for h in range(n_groups):
            @pl.when(c == 0)
            def _():
                pltpu.sync_copy(gin_hbm.at[s, h], gi_v)
                pltpu.sync_copy(sin_hbm.at[s, h], si_v)

            @pl.when(c != 0)
            def _():
                pltpu.sync_copy(gout_hbm.at[s, h], gi_v)
                pltpu.sync_copy(sout_hbm.at[s, h], si_v)

            _gather(0, rows_v, sem_g0)
            _gather(1, rows2_v, sem_g1)
            lax.fori_loop(0, _CG // 2, _pair, 0)

            _drain_rows(sem_s0)
            _drain_rows(sem_s1)
            for w in range(_CG // _CW):
                pltpu.make_async_copy(x_hbm.at[pl.ds(0, _CW)], ones_g,
                                      sem_c).wait()

        plsc.subcore_barrier()

        def _wb(sum_hbm, cnt_hbm):
            @pl.when(s < n_full)
            def _():
                ob = s * wb
                pltpu.sync_copy(acc_sh.at[pl.ds(ob, wb)],
                                sum_hbm.at[pl.ds(ob, wb)])
                pltpu.sync_copy(cnt_sh.at[pl.ds(ob, wb)], cnt_v)
                pltpu.sync_copy(cnt_v, cnt_hbm.at[pl.ds(ob, wb)])

            if wb_tail:
                @pl.when(s == n_full)
                def _():
                    tb = n_full * wb
                    pltpu.sync_copy(acc_sh.at[pl.ds(tb, wb_tail)],
                                    sum_hbm.at[pl.ds(tb, wb_tail)])
                    pltpu.sync_copy(cnt_sh.at[pl.ds(tb, wb_tail)],
                                    cnt_v.at[pl.ds(0, wb_tail)])
                    pltpu.sync_copy(cnt_v.at[pl.ds(0, wb_tail)],
                                    cnt_hbm.at[pl.ds(tb, wb_tail)])

        @pl.when(c == 0)
        def _():
            _wb(sum_in_hbm, cnt_in_hbm)

        @pl.when(c != 0)
        def _():
            _wb(sum_out_hbm, cnt_out_hbm)

    return agg(x, g_in, s_in, g_out, s_out)


def _tc_combine(x, sum_in, cnt_in, sum_out, cnt_out,
                wself_t, wst_t, wts_t, bias_self, bias_st, bias_ts):
    n, d = x.shape
    blk = 1000
    grid = n // blk

    def body(x_r, si_r, ci_r, so_r, co_r, ws_r, wst_r, wts_r,
             bs_r, bst_r, bts_r, o_r):
        rin = (1.0 - _ALPHA) / jnp.maximum(ci_r[...], 1.0)
        rout = _ALPHA / jnp.maximum(co_r[...], 1.0)
        acc = jnp.dot(x_r[...], ws_r[...], preferred_element_type=jnp.float32)
        acc += jnp.dot(si_r[...], wst_r[...],
                       preferred_element_type=jnp.float32) * rin
        acc += jnp.dot(so_r[...], wts_r[...],
                       preferred_element_type=jnp.float32) * rout
        o_r[...] = acc + (bs_r[...] + (1.0 - _ALPHA) * bst_r[...]
                          + _ALPHA * bts_r[...])

    row_spec = pl.BlockSpec((blk, d), lambda i: (i, 0))
    cnt_spec = pl.BlockSpec((blk, 1), lambda i: (i, 0))
    full_spec = pl.BlockSpec((d, d), lambda i: (0, 0))
    bias_spec = pl.BlockSpec((1, d), lambda i: (0, 0))

    return pl.pallas_call(
        body,
        grid=(grid,),
        in_specs=[row_spec, row_spec, cnt_spec, row_spec, cnt_spec,
                  full_spec, full_spec, full_spec,
                  bias_spec, bias_spec, bias_spec],
        out_specs=row_spec,
        out_shape=jax.ShapeDtypeStruct((n, d), jnp.float32),
    )(x, sum_in, cnt_in.reshape(n, 1), sum_out, cnt_out.reshape(n, 1),
      wself_t, wst_t, wts_t,
      bias_self.reshape(1, d), bias_st.reshape(1, d), bias_ts.reshape(1, d))


def kernel(x, edge_index, W_self, b_self, W_st, b_st, W_ts, b_ts):
    n = x.shape[0]
    e = edge_index.shape[1]

    src = edge_index[0].astype(jnp.int32)
    dst = edge_index[1].astype(jnp.int32)

    n_groups = -(-e // (_NS * _CG * _B))
    total = _NS * n_groups * _CG * _B
    acc_rows = -(-(n + 1) // (_NS * _B)) * (_NS * _B)

    def _layout(idx, fill):
        p = jnp.full((total,), fill, jnp.int32)
        p = lax.dynamic_update_slice(p, idx, (0,))
        return p.reshape(_NS, n_groups, _CG, _B)

    g_in = _layout(src, 0)
    s_in = _layout(dst, n)
    g_out = _layout(dst, 0)
    s_out = _layout(src, n)

    sum_in, cnt_in, sum_out, cnt_out = _sc_aggregate(
        x, g_in, s_in, g_out, s_out, n, n_groups, acc_rows)

    return _tc_combine(x, sum_in, cnt_in, sum_out, cnt_out,
                       W_self.T, W_st.T, W_ts.T, b_self, b_st, b_ts)


# speedup vs baseline: 1.8326x; 1.8326x over previous
import functools

import jax
import jax.numpy as jnp
from jax import lax
from jax.experimental import pallas as pl
from jax.experimental.pallas import tpu as pltpu
from jax.experimental.pallas import tpu_sc as plsc

_NC = 2
_NS = 16
_B = 128
_NBUF = 4
_ALPHA = 0.5


def _sc_aggregate(x_bf, g_in, s_in, g_out, s_out, n_nodes, n_chunks,
                  acc_rows):
    d = x_bf.shape[1]
    rows_per_tile = acc_rows // _NS
    wb = -(-(n_nodes // _NS) // _B) * _B
    n_full = n_nodes // wb
    wb_tail = n_nodes - n_full * wb

    mesh = plsc.VectorSubcoreMesh(
        core_axis_name="c", subcore_axis_name="s",
        num_cores=_NC, num_subcores=_NS)

    @functools.partial(
        pl.kernel,
        out_type=[
            jax.ShapeDtypeStruct((n_nodes, d), jnp.bfloat16),
            jax.ShapeDtypeStruct((n_nodes,), jnp.float32),
            jax.ShapeDtypeStruct((n_nodes, d), jnp.bfloat16),
            jax.ShapeDtypeStruct((n_nodes,), jnp.float32),
        ],
        mesh=mesh,
        compiler_params=pltpu.CompilerParams(use_tc_tiling_on_sc=False),
        scratch_types=[
            pltpu.VMEM_SHARED((acc_rows, d), jnp.bfloat16),
            pltpu.VMEM_SHARED((acc_rows,), jnp.float32),
            pltpu.VMEM((n_chunks, _B), jnp.int32),
            pltpu.VMEM((n_chunks, _B), jnp.int32),
            pltpu.VMEM((_B, d), jnp.bfloat16),
            pltpu.VMEM((_B, d), jnp.bfloat16),
            pltpu.VMEM((_B, d), jnp.bfloat16),
            pltpu.VMEM((_B, d), jnp.bfloat16),
            pltpu.VMEM((_B,), jnp.float32),
            pltpu.VMEM((wb,), jnp.float32),
            pltpu.SemaphoreType.DMA,
            pltpu.SemaphoreType.DMA,
            pltpu.SemaphoreType.DMA,
            pltpu.SemaphoreType.DMA,
            pltpu.SemaphoreType.DMA,
            pltpu.SemaphoreType.DMA,
            pltpu.SemaphoreType.DMA,
            pltpu.SemaphoreType.DMA,
            pltpu.SemaphoreType.DMA,
        ],
    )
    def agg(x_hbm, gin_hbm, sin_hbm, gout_hbm, sout_hbm,
            sum_in_hbm, cnt_in_hbm, sum_out_hbm, cnt_out_hbm,
            acc_sh, cnt_sh, gi_v, si_v, r0, r1, r2, r3, ones_v, cnt_v,
            g0, g1, g2, g3, s0, s1, s2, s3, sem_c):
        rows = (r0, r1, r2, r3)
        sem_g = (g0, g1, g2, g3)
        sem_s = (s0, s1, s2, s3)
        c = lax.axis_index("c")
        s = lax.axis_index("s")

        zero2x16 = jnp.zeros((2, 16), jnp.bfloat16)
        zero16 = jnp.zeros((16,), jnp.float32)
        one16 = jnp.ones((16,), jnp.float32)

        def _zero_row(r, carry):
            for k in range(d // 16):
                r0[pl.ds(2 * r, 2), pl.ds(k * 16, 16)] = zero2x16
            return carry
        lax.fori_loop(0, _B // 2, _zero_row, 0)
        for k in range(wb // 16):
            cnt_v[pl.ds(k * 16, 16)] = zero16
        for k in range(_B // 16):
            ones_v[pl.ds(k * 16, 16)] = one16

        def _zero_acc(k, carry):
            pltpu.sync_copy(r0, acc_sh.at[pl.ds(s * rows_per_tile + k * _B,
                                                _B)])
            return carry
        lax.fori_loop(0, rows_per_tile // _B, _zero_acc, 0)
        pltpu.sync_copy(cnt_v, cnt_sh.at[pl.ds(s * wb, wb)])

        @pl.when(c == 0)
        def _():
            pltpu.sync_copy(gin_hbm.at[s], gi_v)
            pltpu.sync_copy(sin_hbm.at[s], si_v)

        @pl.when(c != 0)
        def _():
            pltpu.sync_copy(gout_hbm.at[s], gi_v)
            pltpu.sync_copy(sout_hbm.at[s], si_v)

        plsc.subcore_barrier()

        def _gather(j, b):
            pltpu.async_copy(x_hbm.at[gi_v.at[j]], rows[b], sem_g[b])

        def _drain(b, sem):
            pltpu.make_async_copy(x_hbm.at[gi_v.at[0]], rows[b], sem).wait()

        for b in range(_NBUF):
            _gather(b, b)

        n_rounds = n_chunks // _NBUF

        def _round(q, carry):
            for b in range(_NBUF):
                j = q * _NBUF + b
                _drain(b, sem_g[b])
                pltpu.async_copy(rows[b], acc_sh.at[si_v.at[j]], sem_s[b],
                                 add=True)
                pltpu.async_copy(ones_v, cnt_sh.at[si_v.at[j]], sem_c,
                                 add=True)

                @pl.when(q < n_rounds - 1)
                def _():
                    _drain(b, sem_s[b])
                    _gather(j + _NBUF, b)
            return carry
        lax.fori_loop(0, n_rounds, _round, 0)

        for b in range(_NBUF):
            _drain(b, sem_s[b])
        pltpu.make_async_copy(gin_hbm.at[s], gi_v, sem_c).wait()

        plsc.subcore_barrier()

        def _wb(sum_hbm, cnt_hbm):
            @pl.when(s < n_full)
            def _():
                ob = s * wb
                pltpu.sync_copy(acc_sh.at[pl.ds(ob, wb)],
                                sum_hbm.at[pl.ds(ob, wb)])
                pltpu.sync_copy(cnt_sh.at[pl.ds(ob, wb)], cnt_v)
                pltpu.sync_copy(cnt_v, cnt_hbm.at[pl.ds(ob, wb)])

            if wb_tail:
                @pl.when(s == n_full)
                def _():
                    tb = n_full * wb
                    pltpu.sync_copy(acc_sh.at[pl.ds(tb, wb_tail)],
                                    sum_hbm.at[pl.ds(tb, wb_tail)])
                    pltpu.sync_copy(cnt_sh.at[pl.ds(tb, wb_tail)],
                                    cnt_v.at[pl.ds(0, wb_tail)])
                    pltpu.sync_copy(cnt_v.at[pl.ds(0, wb_tail)],
                                    cnt_hbm.at[pl.ds(tb, wb_tail)])

        @pl.when(c == 0)
        def _():
            _wb(sum_in_hbm, cnt_in_hbm)

        @pl.when(c != 0)
        def _():
            _wb(sum_out_hbm, cnt_out_hbm)

    return agg(x_bf, g_in, s_in, g_out, s_out)


def _tc_combine(x, sum_in, cnt_in, sum_out, cnt_out,
                wself_t, wst_t, wts_t, bias_self, bias_st, bias_ts):
    n, d = x.shape
    blk = 1000
    grid = n // blk

    def body(x_r, si_r, ci_r, so_r, co_r, ws_r, wst_r, wts_r,
             bs_r, bst_r, bts_r, o_r):
        rin = (1.0 - _ALPHA) / jnp.maximum(ci_r[...], 1.0)
        rout = _ALPHA / jnp.maximum(co_r[...], 1.0)
        acc = jnp.dot(x_r[...], ws_r[...], preferred_element_type=jnp.float32)
        acc += jnp.dot(si_r[...].astype(jnp.float32), wst_r[...],
                       preferred_element_type=jnp.float32) * rin
        acc += jnp.dot(so_r[...].astype(jnp.float32), wts_r[...],
                       preferred_element_type=jnp.float32) * rout
        o_r[...] = acc + (bs_r[...] + (1.0 - _ALPHA) * bst_r[...]
                          + _ALPHA * bts_r[...])

    row_spec = pl.BlockSpec((blk, d), lambda i: (i, 0))
    cnt_spec = pl.BlockSpec((blk, 1), lambda i: (i, 0))
    full_spec = pl.BlockSpec((d, d), lambda i: (0, 0))
    bias_spec = pl.BlockSpec((1, d), lambda i: (0, 0))

    return pl.pallas_call(
        body,
        grid=(grid,),
        in_specs=[row_spec, row_spec, cnt_spec, row_spec, cnt_spec,
                  full_spec, full_spec, full_spec,
                  bias_spec, bias_spec, bias_spec],
        out_specs=row_spec,
        out_shape=jax.ShapeDtypeStruct((n, d), jnp.float32),
    )(x, sum_in, cnt_in.reshape(n, 1), sum_out, cnt_out.reshape(n, 1),
      wself_t, wst_t, wts_t,
      bias_self.reshape(1, d), bias_st.reshape(1, d), bias_ts.reshape(1, d))


def kernel(x, edge_index, W_self, b_self, W_st, b_st, W_ts, b_ts):
    n = x.shape[0]
    e = edge_index.shape[1]

    src = edge_index[0].astype(jnp.int32)
    dst = edge_index[1].astype(jnp.int32)
    x_bf = x.astype(jnp.bfloat16)

    n_chunks = -(-e // (_NS * _B * _NBUF)) * _NBUF
    total = _NS * n_chunks * _B
    acc_rows = -(-(n + 1) // (_NS * _B)) * (_NS * _B)

    def _layout(idx, fill):
        p = jnp.full((total,), fill, jnp.int32)
        p = lax.dynamic_update_slice(p, idx, (0,))
        return p.reshape(_NS, n_chunks, _B)

    g_in = _layout(src, 0)
    s_in = _layout(dst, n)
    g_out = _layout(dst, 0)
    s_out = _layout(src, n)

    sum_in, cnt_in, sum_out, cnt_out = _sc_aggregate(
        x_bf, g_in, s_in, g_out, s_out, n, n_chunks, acc_rows)

    return _tc_combine(x, sum_in, cnt_in, sum_out, cnt_out,
                       W_self.T, W_st.T, W_ts.T, b_self, b_st, b_ts)


# speedup vs baseline: 82.0502x; 44.7737x over previous
import functools

import jax
import jax.numpy as jnp
from jax import lax
from jax.experimental import pallas as pl
from jax.experimental.pallas import tpu as pltpu
from jax.experimental.pallas import tpu_sc as plsc

_NC = 2
_NS = 16
_B = 128
_CG = 40
_ALPHA = 0.5


def _sc_aggregate(x_bf, g_in, s_in, g_out, s_out, n_nodes, n_groups,
                  acc_rows):
    d = x_bf.shape[1]
    rows_per_tile = acc_rows // _NS
    wb = -(-(n_nodes // _NS) // _B) * _B
    n_full = n_nodes // wb
    wb_tail = n_nodes - n_full * wb

    mesh = plsc.VectorSubcoreMesh(
        core_axis_name="c", subcore_axis_name="s",
        num_cores=_NC, num_subcores=_NS)

    @functools.partial(
        pl.kernel,
        out_type=[
            jax.ShapeDtypeStruct((n_nodes, d), jnp.bfloat16),
            jax.ShapeDtypeStruct((n_nodes,), jnp.float32),
            jax.ShapeDtypeStruct((n_nodes, d), jnp.bfloat16),
            jax.ShapeDtypeStruct((n_nodes,), jnp.float32),
        ],
        mesh=mesh,
        compiler_params=pltpu.CompilerParams(use_tc_tiling_on_sc=False),
        scratch_types=[
            pltpu.VMEM_SHARED((acc_rows, d), jnp.bfloat16),
            pltpu.VMEM_SHARED((n_nodes, d), jnp.bfloat16),
            pltpu.VMEM_SHARED((acc_rows,), jnp.float32),
            pltpu.VMEM((_CG, _B), jnp.int32),
            pltpu.VMEM((_CG, _B), jnp.int32),
            pltpu.VMEM((_B, d), jnp.bfloat16),
            pltpu.VMEM((_B, d), jnp.bfloat16),
            pltpu.VMEM((_B,), jnp.float32),
            pltpu.VMEM((wb,), jnp.float32),
            pltpu.SemaphoreType.DMA,
            pltpu.SemaphoreType.DMA,
            pltpu.SemaphoreType.DMA,
            pltpu.SemaphoreType.DMA,
            pltpu.SemaphoreType.DMA,
        ],
    )
    def agg(x_hbm, gin_hbm, sin_hbm, gout_hbm, sout_hbm,
            sum_in_hbm, cnt_in_hbm, sum_out_hbm, cnt_out_hbm,
            acc_sh, x_sh, cnt_sh, gi_v, si_v, r0, r1, ones_v, cnt_v,
            g0, g1, s0, s1, sem_c):
        rows = (r0, r1)
        sem_g = (g0, g1)
        sem_s = (s0, s1)
        c = lax.axis_index("c")
        s = lax.axis_index("s")

        @pl.when(s < n_full)
        def _():
            ob = s * wb
            pltpu.sync_copy(x_hbm.at[pl.ds(ob, wb)], x_sh.at[pl.ds(ob, wb)])

        if wb_tail:
            @pl.when(s == n_full)
            def _():
                tb = n_full * wb
                pltpu.sync_copy(x_hbm.at[pl.ds(tb, wb_tail)],
                                x_sh.at[pl.ds(tb, wb_tail)])

        zero2x16 = jnp.zeros((2, 16), jnp.bfloat16)
        zero16 = jnp.zeros((16,), jnp.float32)
        one16 = jnp.ones((16,), jnp.float32)

        def _zero_row(r, carry):
            for k in range(d // 16):
                r0[pl.ds(2 * r, 2), pl.ds(k * 16, 16)] = zero2x16
            return carry
        lax.fori_loop(0, _B // 2, _zero_row, 0)
        for k in range(wb // 16):
            cnt_v[pl.ds(k * 16, 16)] = zero16
        for k in range(_B // 16):
            ones_v[pl.ds(k * 16, 16)] = one16

        def _zero_acc(k, carry):
            pltpu.sync_copy(r0, acc_sh.at[pl.ds(s * rows_per_tile + k * _B,
                                                _B)])
            return carry
        lax.fori_loop(0, rows_per_tile // _B, _zero_acc, 0)
        pltpu.sync_copy(cnt_v, cnt_sh.at[pl.ds(s * wb, wb)])

        plsc.subcore_barrier()

        def _gather(j, b):
            pltpu.async_copy(x_sh.at[gi_v.at[j]], rows[b], sem_g[b])

        def _drain(b, sem):
            pltpu.make_async_copy(x_hbm.at[gi_v.at[0]], rows[b], sem).wait()

        def _pair(p, carry):
            for b in range(2):
                j = 2 * p + b
                _drain(b, sem_g[b])
                pltpu.async_copy(rows[b], acc_sh.at[si_v.at[j]], sem_s[b],
                                 add=True)
                pltpu.async_copy(ones_v, cnt_sh.at[si_v.at[j]], sem_c,
                                 add=True)

                @pl.when(p < _CG // 2 - 1)
                def _():
                    _drain(b, sem_s[b])
                    _gather(j + 2, b)
            return carry

        for h in range(n_groups):
            @pl.when(c == 0)
            def _():
                pltpu.sync_copy(gin_hbm.at[s, h], gi_v)
                pltpu.sync_copy(sin_hbm.at[s, h], si_v)

            @pl.when(c != 0)
            def _():
                pltpu.sync_copy(gout_hbm.at[s, h], gi_v)
                pltpu.sync_copy(sout_hbm.at[s, h], si_v)

            _gather(0, 0)
            _gather(1, 1)
            lax.fori_loop(0, _CG // 2, _pair, 0)

            _drain(0, sem_s[0])
            _drain(1, sem_s[1])
            pltpu.make_async_copy(gin_hbm.at[s, h], gi_v, sem_c).wait()

        plsc.subcore_barrier()

        def _wb(sum_hbm, cnt_hbm):
            @pl.when(s < n_full)
            def _():
                ob = s * wb
                pltpu.sync_copy(acc_sh.at[pl.ds(ob, wb)],
                                sum_hbm.at[pl.ds(ob, wb)])
                pltpu.sync_copy(cnt_sh.at[pl.ds(ob, wb)], cnt_v)
                pltpu.sync_copy(cnt_v, cnt_hbm.at[pl.ds(ob, wb)])

            if wb_tail:
                @pl.when(s == n_full)
                def _():
                    tb = n_full * wb
                    pltpu.sync_copy(acc_sh.at[pl.ds(tb, wb_tail)],
                                    sum_hbm.at[pl.ds(tb, wb_tail)])
                    pltpu.sync_copy(cnt_sh.at[pl.ds(tb, wb_tail)],
                                    cnt_v.at[pl.ds(0, wb_tail)])
                    pltpu.sync_copy(cnt_v.at[pl.ds(0, wb_tail)],
                                    cnt_hbm.at[pl.ds(tb, wb_tail)])

        @pl.when(c == 0)
        def _():
            _wb(sum_in_hbm, cnt_in_hbm)

        @pl.when(c != 0)
        def _():
            _wb(sum_out_hbm, cnt_out_hbm)

    return agg(x_bf, g_in, s_in, g_out, s_out)


def _tc_combine(x, sum_in, cnt_in, sum_out, cnt_out,
                wself_t, wst_t, wts_t, bias_self, bias_st, bias_ts):
    n, d = x.shape
    blk = 1000
    grid = n // blk

    def body(x_r, si_r, ci_r, so_r, co_r, ws_r, wst_r, wts_r,
             bs_r, bst_r, bts_r, o_r):
        rin = (1.0 - _ALPHA) / jnp.maximum(ci_r[...], 1.0)
        rout = _ALPHA / jnp.maximum(co_r[...], 1.0)
        acc = jnp.dot(x_r[...], ws_r[...], preferred_element_type=jnp.float32)
        acc += jnp.dot(si_r[...].astype(jnp.float32), wst_r[...],
                       preferred_element_type=jnp.float32) * rin
        acc += jnp.dot(so_r[...].astype(jnp.float32), wts_r[...],
                       preferred_element_type=jnp.float32) * rout
        o_r[...] = acc + (bs_r[...] + (1.0 - _ALPHA) * bst_r[...]
                          + _ALPHA * bts_r[...])

    row_spec = pl.BlockSpec((blk, d), lambda i: (i, 0))
    cnt_spec = pl.BlockSpec((blk, 1), lambda i: (i, 0))
    full_spec = pl.BlockSpec((d, d), lambda i: (0, 0))
    bias_spec = pl.BlockSpec((1, d), lambda i: (0, 0))

    return pl.pallas_call(
        body,
        grid=(grid,),
        in_specs=[row_spec, row_spec, cnt_spec, row_spec, cnt_spec,
                  full_spec, full_spec, full_spec,
                  bias_spec, bias_spec, bias_spec],
        out_specs=row_spec,
        out_shape=jax.ShapeDtypeStruct((n, d), jnp.float32),
    )(x, sum_in, cnt_in.reshape(n, 1), sum_out, cnt_out.reshape(n, 1),
      wself_t, wst_t, wts_t,
      bias_self.reshape(1, d), bias_st.reshape(1, d), bias_ts.reshape(1, d))


def kernel(x, edge_index, W_self, b_self, W_st, b_st, W_ts, b_ts):
    n = x.shape[0]
    e = edge_index.shape[1]

    src = edge_index[0].astype(jnp.int32)
    dst = edge_index[1].astype(jnp.int32)
    x_bf = x.astype(jnp.bfloat16)

    n_groups = -(-e // (_NS * _CG * _B))
    total = _NS * n_groups * _CG * _B
    acc_rows = -(-(n + 1) // (_NS * _B)) * (_NS * _B)

    def _layout(idx, fill):
        p = jnp.full((total,), fill, jnp.int32)
        p = lax.dynamic_update_slice(p, idx, (0,))
        return p.reshape(_NS, n_groups, _CG, _B)

    g_in = _layout(src, 0)
    s_in = _layout(dst, n)
    g_out = _layout(dst, 0)
    s_out = _layout(src, n)

    sum_in, cnt_in, sum_out, cnt_out = _sc_aggregate(
        x_bf, g_in, s_in, g_out, s_out, n, n_groups, acc_rows)

    return _tc_combine(x, sum_in, cnt_in, sum_out, cnt_out,
                       W_self.T, W_st.T, W_ts.T, b_self, b_st, b_ts)
